# baseline (speedup 1.0000x reference)
import functools

import jax
import jax.numpy as jnp
from jax import lax
from jax.experimental import pallas as pl
from jax.experimental.pallas import tpu as pltpu
from jax.experimental.pallas import tpu_sc as plsc

N = 10000
D = 128
NCH = 160
C = 64
EPT = NCH * C
NW = 32
EPAD = NW * EPT
NPAD = 10240
RPT = NPAD // 16

_mesh = plsc.VectorSubcoreMesh(core_axis_name="c", subcore_axis_name="s")


@functools.partial(
    pl.kernel,
    out_type=jax.ShapeDtypeStruct((2, NPAD), jnp.float32),
    mesh=_mesh,
    scratch_types=[
        pltpu.VMEM((NCH, C), jnp.int32),
        pltpu.VMEM((NPAD,), jnp.float32),
        pltpu.VMEM((RPT,), jnp.float32),
        pltpu.VMEM((RPT,), jnp.float32),
        pltpu.VMEM_SHARED((16, NPAD), jnp.float32),
        pltpu.SemaphoreType.DMA,
    ],
    compiler_params=pltpu.CompilerParams(needs_layout_passes=False),
)
def _sc_deg(dst_hbm, out_hbm, dst_v, hist, red, tmp, shared, sem):
    cid = lax.axis_index("c")
    sid = lax.axis_index("s")
    wid = sid * 2 + cid
    pltpu.sync_copy(dst_hbm.at[wid], dst_v)
    zero = jnp.zeros((16,), jnp.float32)
    ones = jnp.ones((16,), jnp.float32)

    def z(i, carry):
        hist[pl.ds(i * 16, 16)] = zero
        return carry

    lax.fori_loop(0, NPAD // 16, z, None)

    def body(j, carry):
        for k in range(C // 16):
            idx = dst_v[j, pl.ds(k * 16, 16)]
            plsc.addupdate_scatter(hist, [idx], ones)
        return carry

    lax.fori_loop(0, NCH, body, None)

    pltpu.sync_copy(hist, shared.at[sid])
    plsc.subcore_barrier()

    def zr(i, carry):
        red[pl.ds(i * 16, 16)] = zero
        return carry

    lax.fori_loop(0, RPT // 16, zr, None)

    def acc_t(t, carry):
        pltpu.sync_copy(shared.at[t, pl.ds(sid * RPT, RPT)], tmp)

        def addv(i, c2):
            s = pl.ds(i * 16, 16)
            red[s] = red[s] + tmp[s]
            return c2

        lax.fori_loop(0, RPT // 16, addv, None)
        return carry

    lax.fori_loop(0, 16, acc_t, None)
    pltpu.sync_copy(red, out_hbm.at[cid, pl.ds(sid * RPT, RPT)])


K = 4
UB = 8
NH = NCH // 4


@functools.partial(
    pl.kernel,
    out_type=jax.ShapeDtypeStruct((2, NPAD, D), jnp.float32),
    mesh=_mesh,
    scratch_types=[
        pltpu.VMEM((NH, C), jnp.int32),
        pltpu.VMEM((NH, C), jnp.int32),
        [pltpu.VMEM((C, D), jnp.float32) for _ in range(K)],
        pltpu.VMEM_SHARED((NPAD, D), jnp.float32),
        [pltpu.SemaphoreType.DMA for _ in range(K)],
        [pltpu.SemaphoreType.DMA for _ in range(K)],
    ],
)
def _sc_scatter(g_hbm, src_hbm, dst_hbm, out_hbm, src_v, dst_v, rows, acc,
                gsem, ssem):
    cid = lax.axis_index("c")
    sid = lax.axis_index("s")
    wid = sid * 2 + cid
    zero = jnp.zeros((16,), jnp.float32)

    def zrow(r, carry):
        for k in range(D // 16):
            rows[0][r, pl.ds(k * 16, 16)] = zero
        return carry

    lax.fori_loop(0, C, zrow, None)
    for k in range(RPT // C):
        pltpu.sync_copy(rows[0], acc.at[pl.ds(sid * RPT + k * C, C)])
    plsc.subcore_barrier()

    for h in range(4):
        pltpu.sync_copy(src_hbm.at[wid, pl.ds(h * NH, NH)], src_v)
        pltpu.sync_copy(dst_hbm.at[wid, pl.ds(h * NH, NH)], dst_v)

        def body(i, carry):
            j = i * UB
            gd = {b: pltpu.async_copy(g_hbm.at[src_v.at[j + b]], rows[b],
                                      gsem[b]) for b in range(2)}
            sd = {}
            for u in range(UB):
                b = u % K
                gd[u].wait()
                sd[u] = pltpu.async_copy(rows[b], acc.at[dst_v.at[j + u]],
                                         ssem[b], add=True)
                if u + 2 < UB:
                    nb = (u + 2) % K
                    if u - 2 >= 0:
                        sd[u - 2].wait()
                    gd[u + 2] = pltpu.async_copy(
                        g_hbm.at[src_v.at[j + u + 2]], rows[nb], gsem[nb])
            sd[UB - 2].wait()
            sd[UB - 1].wait()
            return carry

        lax.fori_loop(0, NH // UB, body, None)
    plsc.subcore_barrier()
    pltpu.sync_copy(acc.at[pl.ds(sid * RPT, RPT)],
                    out_hbm.at[cid, pl.ds(sid * RPT, RPT)])


def _elu(v):
    return jnp.where(v > 0.0, v, jnp.exp(jnp.minimum(v, 0.0)) - 1.0)


_MB = 2000
_GRID = N // _MB


def _tc_pre_body(degp_ref, x_ref, w_ref, dis_ref, g_ref):
    deg = degp_ref[0] + degp_ref[1] + 1.0
    dis = lax.rsqrt(deg)
    dis_ref[...] = dis
    h = jnp.dot(x_ref[...], w_ref[...], preferred_element_type=jnp.float32)
    g_ref[...] = h * dis


def _tc_pre(degp, x, w):
    return pl.pallas_call(
        _tc_pre_body,
        grid=(_GRID,),
        in_specs=[
            pl.BlockSpec((2, _MB, 1), lambda i: (0, i, 0)),
            pl.BlockSpec((_MB, D), lambda i: (i, 0)),
            pl.BlockSpec((D, D), lambda i: (0, 0)),
        ],
        out_specs=[
            pl.BlockSpec((_MB, 1), lambda i: (i, 0)),
            pl.BlockSpec((_MB, D), lambda i: (i, 0)),
        ],
        out_shape=[
            jax.ShapeDtypeStruct((N, 1), jnp.float32),
            jax.ShapeDtypeStruct((N, D), jnp.float32),
        ],
    )(degp, x, w)


def _tc_mid_body(p_ref, g_ref, dis_ref, b_ref, w_ref, out_ref):
    s = p_ref[0] + p_ref[1] + g_ref[...]
    h = _elu(dis_ref[...] * s + b_ref[...])
    out_ref[...] = jnp.dot(h, w_ref[...],
                           preferred_element_type=jnp.float32) * dis_ref[...]


def _tc_mid(p, g, dis, b, w):
    return pl.pallas_call(
        _tc_mid_body,
        grid=(_GRID,),
        in_specs=[
            pl.BlockSpec((2, _MB, D), lambda i: (0, i, 0)),
            pl.BlockSpec((_MB, D), lambda i: (i, 0)),
            pl.BlockSpec((_MB, 1), lambda i: (i, 0)),
            pl.BlockSpec((1, D), lambda i: (0, 0)),
            pl.BlockSpec((D, D), lambda i: (0, 0)),
        ],
        out_specs=pl.BlockSpec((_MB, D), lambda i: (i, 0)),
        out_shape=jax.ShapeDtypeStruct((N, D), jnp.float32),
    )(p, g, dis, b, w)


def _tc_post_body(p_ref, g_ref, dis_ref, b_ref, out_ref):
    s = p_ref[0] + p_ref[1] + g_ref[...]
    out_ref[...] = _elu(dis_ref[...] * s + b_ref[...])


def _tc_post(p, g, dis, b):
    return pl.pallas_call(
        _tc_post_body,
        grid=(_GRID,),
        in_specs=[
            pl.BlockSpec((2, _MB, D), lambda i: (0, i, 0)),
            pl.BlockSpec((_MB, D), lambda i: (i, 0)),
            pl.BlockSpec((_MB, 1), lambda i: (i, 0)),
            pl.BlockSpec((1, D), lambda i: (0, 0)),
        ],
        out_specs=pl.BlockSpec((_MB, D), lambda i: (i, 0)),
        out_shape=jax.ShapeDtypeStruct((N, D), jnp.float32),
    )(p, g, dis, b)


def kernel(x, edge_index, W1, b1, W2, b2):
    e = edge_index.astype(jnp.int32)
    npad = EPAD - e.shape[1]
    pad_src = jnp.arange(npad, dtype=jnp.int32) % N
    src3 = jnp.concatenate([e[0], pad_src]).reshape(NW, NCH, C)
    pad_dst = N + (jnp.arange(npad, dtype=jnp.int32) % (NPAD - N))
    dst3 = jnp.concatenate([e[1], pad_dst]).reshape(NW, NCH, C)

    degp = _sc_deg(dst3).reshape(2, NPAD, 1)
    dis, g1 = _tc_pre(degp, x, W1)
    p1 = _sc_scatter(g1, src3, dst3)
    g2 = _tc_mid(p1, g1, dis, b1.reshape(1, D), W2)
    p2 = _sc_scatter(g2, src3, dst3)
    return _tc_post(p2, g2, dis, b2.reshape(1, D))


# speedup vs baseline: 1.0610x; 1.0610x over previous
import functools

import jax
import jax.numpy as jnp
from jax import lax
from jax.experimental import pallas as pl
from jax.experimental.pallas import tpu as pltpu
from jax.experimental.pallas import tpu_sc as plsc

N = 10000
D = 128
NCH = 160
C = 64
EPT = NCH * C
NW = 32
EPAD = NW * EPT
NPAD = 10240
RPT = NPAD // 16

_mesh = plsc.VectorSubcoreMesh(core_axis_name="c", subcore_axis_name="s")


@functools.partial(
    pl.kernel,
    out_type=jax.ShapeDtypeStruct((2, NPAD), jnp.float32),
    mesh=_mesh,
    scratch_types=[
        pltpu.VMEM((NCH, C), jnp.int32),
        pltpu.VMEM((NPAD,), jnp.float32),
        pltpu.VMEM((RPT,), jnp.float32),
        pltpu.VMEM((RPT,), jnp.float32),
        pltpu.VMEM_SHARED((16, NPAD), jnp.float32),
        pltpu.SemaphoreType.DMA,
    ],
    compiler_params=pltpu.CompilerParams(needs_layout_passes=False),
)
def _sc_deg(dst_hbm, out_hbm, dst_v, hist, red, tmp, shared, sem):
    cid = lax.axis_index("c")
    sid = lax.axis_index("s")
    wid = sid * 2 + cid
    pltpu.sync_copy(dst_hbm.at[wid], dst_v)
    zero = jnp.zeros((16,), jnp.float32)
    ones = jnp.ones((16,), jnp.float32)

    def z(i, carry):
        hist[pl.ds(i * 16, 16)] = zero
        return carry

    lax.fori_loop(0, NPAD // 16, z, None)

    def body(j, carry):
        for k in range(C // 16):
            idx = dst_v[j, pl.ds(k * 16, 16)]
            plsc.addupdate_scatter(hist, [idx], ones)
        return carry

    lax.fori_loop(0, NCH, body, None)

    pltpu.sync_copy(hist, shared.at[sid])
    plsc.subcore_barrier()

    def zr(i, carry):
        red[pl.ds(i * 16, 16)] = zero
        return carry

    lax.fori_loop(0, RPT // 16, zr, None)

    def acc_t(t, carry):
        pltpu.sync_copy(shared.at[t, pl.ds(sid * RPT, RPT)], tmp)

        def addv(i, c2):
            s = pl.ds(i * 16, 16)
            red[s] = red[s] + tmp[s]
            return c2

        lax.fori_loop(0, RPT // 16, addv, None)
        return carry

    lax.fori_loop(0, 16, acc_t, None)
    pltpu.sync_copy(red, out_hbm.at[cid, pl.ds(sid * RPT, RPT)])


K = 4
UB = 8
NH = NCH // 4


@functools.partial(
    pl.kernel,
    out_type=jax.ShapeDtypeStruct((2, NPAD, D), jnp.float32),
    mesh=_mesh,
    scratch_types=[
        pltpu.VMEM((NH, C), jnp.int32),
        pltpu.VMEM((NH, C), jnp.int32),
        [pltpu.VMEM((C, D), jnp.float32) for _ in range(K)],
        pltpu.VMEM_SHARED((NPAD, D), jnp.float32),
        [pltpu.SemaphoreType.DMA for _ in range(K)],
        [pltpu.SemaphoreType.DMA for _ in range(K)],
    ],
)
def _sc_scatter(g_hbm, src_hbm, dst_hbm, out_hbm, src_v, dst_v, rows, acc,
                gsem, ssem):
    cid = lax.axis_index("c")
    sid = lax.axis_index("s")
    wid = sid * 2 + cid
    zero = jnp.zeros((16,), jnp.float32)

    def zrow(r, carry):
        for k in range(D // 16):
            rows[0][r, pl.ds(k * 16, 16)] = zero
        return carry

    lax.fori_loop(0, C, zrow, None)
    for k in range(RPT // C):
        pltpu.sync_copy(rows[0], acc.at[pl.ds(sid * RPT + k * C, C)])
    plsc.subcore_barrier()

    for h in range(4):
        pltpu.sync_copy(src_hbm.at[wid, pl.ds(h * NH, NH)], src_v)
        pltpu.sync_copy(dst_hbm.at[wid, pl.ds(h * NH, NH)], dst_v)

        def body(i, carry):
            j = i * UB
            gd = {b: pltpu.async_copy(g_hbm.at[src_v.at[j + b]], rows[b],
                                      gsem[b]) for b in range(2)}
            for u in range(UB):
                b = u % K
                gd[u].wait()
                if u + 2 < UB:
                    gd[u + 2] = pltpu.async_copy(
                        g_hbm.at[src_v.at[j + u + 2]], rows[(u + 2) % K],
                        gsem[(u + 2) % K])
                pltpu.sync_copy(rows[b], acc.at[dst_v.at[j + u]], add=True)
            return carry

        lax.fori_loop(0, NH // UB, body, None)
    plsc.subcore_barrier()
    pltpu.sync_copy(acc.at[pl.ds(sid * RPT, RPT)],
                    out_hbm.at[cid, pl.ds(sid * RPT, RPT)])


def _elu(v):
    return jnp.where(v > 0.0, v, jnp.exp(jnp.minimum(v, 0.0)) - 1.0)


_MB = 2000
_GRID = N // _MB


def _tc_pre_body(degp_ref, x_ref, w_ref, dis_ref, g_ref):
    deg = degp_ref[0] + degp_ref[1] + 1.0
    dis = lax.rsqrt(deg)
    dis_ref[...] = dis
    h = jnp.dot(x_ref[...], w_ref[...], preferred_element_type=jnp.float32)
    g_ref[...] = h * dis


def _tc_pre(degp, x, w):
    return pl.pallas_call(
        _tc_pre_body,
        grid=(_GRID,),
        in_specs=[
            pl.BlockSpec((2, _MB, 1), lambda i: (0, i, 0)),
            pl.BlockSpec((_MB, D), lambda i: (i, 0)),
            pl.BlockSpec((D, D), lambda i: (0, 0)),
        ],
        out_specs=[
            pl.BlockSpec((_MB, 1), lambda i: (i, 0)),
            pl.BlockSpec((_MB, D), lambda i: (i, 0)),
        ],
        out_shape=[
            jax.ShapeDtypeStruct((N, 1), jnp.float32),
            jax.ShapeDtypeStruct((N, D), jnp.float32),
        ],
    )(degp, x, w)


def _tc_mid_body(p_ref, g_ref, dis_ref, b_ref, w_ref, out_ref):
    s = p_ref[0] + p_ref[1] + g_ref[...]
    h = _elu(dis_ref[...] * s + b_ref[...])
    out_ref[...] = jnp.dot(h, w_ref[...],
                           preferred_element_type=jnp.float32) * dis_ref[...]


def _tc_mid(p, g, dis, b, w):
    return pl.pallas_call(
        _tc_mid_body,
        grid=(_GRID,),
        in_specs=[
            pl.BlockSpec((2, _MB, D), lambda i: (0, i, 0)),
            pl.BlockSpec((_MB, D), lambda i: (i, 0)),
            pl.BlockSpec((_MB, 1), lambda i: (i, 0)),
            pl.BlockSpec((1, D), lambda i: (0, 0)),
            pl.BlockSpec((D, D), lambda i: (0, 0)),
        ],
        out_specs=pl.BlockSpec((_MB, D), lambda i: (i, 0)),
        out_shape=jax.ShapeDtypeStruct((N, D), jnp.float32),
    )(p, g, dis, b, w)


def _tc_post_body(p_ref, g_ref, dis_ref, b_ref, out_ref):
    s = p_ref[0] + p_ref[1] + g_ref[...]
    out_ref[...] = _elu(dis_ref[...] * s + b_ref[...])


def _tc_post(p, g, dis, b):
    return pl.pallas_call(
        _tc_post_body,
        grid=(_GRID,),
        in_specs=[
            pl.BlockSpec((2, _MB, D), lambda i: (0, i, 0)),
            pl.BlockSpec((_MB, D), lambda i: (i, 0)),
            pl.BlockSpec((_MB, 1), lambda i: (i, 0)),
            pl.BlockSpec((1, D), lambda i: (0, 0)),
        ],
        out_specs=pl.BlockSpec((_MB, D), lambda i: (i, 0)),
        out_shape=jax.ShapeDtypeStruct((N, D), jnp.float32),
    )(p, g, dis, b)


def kernel(x, edge_index, W1, b1, W2, b2):
    e = edge_index.astype(jnp.int32)
    npad = EPAD - e.shape[1]
    pad_src = jnp.arange(npad, dtype=jnp.int32) % N
    src3 = jnp.concatenate([e[0], pad_src]).reshape(NW, NCH, C)
    pad_dst = N + (jnp.arange(npad, dtype=jnp.int32) % (NPAD - N))
    dst3 = jnp.concatenate([e[1], pad_dst]).reshape(NW, NCH, C)

    degp = _sc_deg(dst3).reshape(2, NPAD, 1)
    dis, g1 = _tc_pre(degp, x, W1)
    p1 = _sc_scatter(g1, src3, dst3)
    g2 = _tc_mid(p1, g1, dis, b1.reshape(1, D), W2)
    p2 = _sc_scatter(g2, src3, dst3)
    return _tc_post(p2, g2, dis, b2.reshape(1, D))


# speedup vs baseline: 1.2133x; 1.1435x over previous
import functools

import jax
import jax.numpy as jnp
from jax import lax
from jax.experimental import pallas as pl
from jax.experimental.pallas import tpu as pltpu
from jax.experimental.pallas import tpu_sc as plsc

N = 10000
D = 128
NCH = 80
C = 125
EPT = NCH * C
NW = 32
NPAD = 10240
RPT = NPAD // 16

_mesh = plsc.VectorSubcoreMesh(core_axis_name="c", subcore_axis_name="s")


@functools.partial(
    pl.kernel,
    out_type=jax.ShapeDtypeStruct((2, NPAD), jnp.float32),
    mesh=_mesh,
    scratch_types=[
        pltpu.VMEM((EPT,), jnp.int32),
        pltpu.VMEM((NPAD,), jnp.float32),
        pltpu.VMEM((RPT,), jnp.float32),
        pltpu.VMEM((RPT,), jnp.float32),
        pltpu.VMEM_SHARED((16, NPAD), jnp.float32),
        pltpu.SemaphoreType.DMA,
    ],
    compiler_params=pltpu.CompilerParams(needs_layout_passes=False),
)
def _sc_deg(dst_hbm, out_hbm, dst_v, hist, red, tmp, shared, sem):
    cid = lax.axis_index("c")
    sid = lax.axis_index("s")
    wid = sid * 2 + cid
    pltpu.sync_copy(dst_hbm.at[wid], dst_v)
    zero = jnp.zeros((16,), jnp.float32)
    ones = jnp.ones((16,), jnp.float32)

    def z(i, carry):
        hist[pl.ds(i * 16, 16)] = zero
        return carry

    lax.fori_loop(0, NPAD // 16, z, None)

    def body(j, carry):
        idx = dst_v[pl.ds(j * 16, 16)]
        plsc.addupdate_scatter(hist, [idx], ones)
        return carry

    lax.fori_loop(0, EPT // 16, body, None)

    pltpu.sync_copy(hist, shared.at[sid])
    plsc.subcore_barrier()

    def zr(i, carry):
        red[pl.ds(i * 16, 16)] = zero
        return carry

    lax.fori_loop(0, RPT // 16, zr, None)

    def acc_t(t, carry):
        pltpu.sync_copy(shared.at[t, pl.ds(sid * RPT, RPT)], tmp)

        def addv(i, c2):
            s = pl.ds(i * 16, 16)
            red[s] = red[s] + tmp[s]
            return c2

        lax.fori_loop(0, RPT // 16, addv, None)
        return carry

    lax.fori_loop(0, 16, acc_t, None)
    pltpu.sync_copy(red, out_hbm.at[cid, pl.ds(sid * RPT, RPT)])


K = 2
UB = 8
NH = NCH // 2


@functools.partial(
    pl.kernel,
    out_type=jax.ShapeDtypeStruct((2, NPAD, D), jnp.float32),
    mesh=_mesh,
    scratch_types=[
        pltpu.VMEM((NH, C), jnp.int32),
        pltpu.VMEM((NH, C), jnp.int32),
        [pltpu.VMEM((C, D), jnp.float32) for _ in range(K)],
        pltpu.VMEM_SHARED((NPAD, D), jnp.float32),
        [pltpu.SemaphoreType.DMA for _ in range(K)],
    ],
)
def _sc_scatter(g_hbm, src_hbm, dst_hbm, out_hbm, src_v, dst_v, rows, acc,
                gsem):
    cid = lax.axis_index("c")
    sid = lax.axis_index("s")
    wid = sid * 2 + cid
    zero = jnp.zeros((16,), jnp.float32)

    def zrow(r, carry):
        for k in range(D // 16):
            rows[0][r, pl.ds(k * 16, 16)] = zero
        return carry

    lax.fori_loop(0, C, zrow, None)
    for k in range(RPT // C):
        pltpu.sync_copy(rows[0], acc.at[pl.ds(sid * RPT + k * C, C)])
    rem = RPT - (RPT // C) * C
    if rem:
        pltpu.sync_copy(rows[0].at[pl.ds(0, rem)],
                        acc.at[pl.ds(sid * RPT + (RPT // C) * C, rem)])
    plsc.subcore_barrier()

    for h in range(2):
        pltpu.sync_copy(src_hbm.at[wid, pl.ds(h * NH, NH)], src_v)
        pltpu.sync_copy(dst_hbm.at[wid, pl.ds(h * NH, NH)], dst_v)

        def body(i, carry):
            j = i * UB
            gd = {b: pltpu.async_copy(g_hbm.at[src_v.at[j + b]], rows[b],
                                      gsem[b]) for b in range(K)}
            for u in range(UB):
                b = u % K
                gd[u].wait()
                if u + K < UB:
                    gd[u + K] = pltpu.async_copy(
                        g_hbm.at[src_v.at[j + u + K]], rows[b], gsem[b])
                pltpu.sync_copy(rows[b], acc.at[dst_v.at[j + u]], add=True)
            return carry

        lax.fori_loop(0, NH // UB, body, None)
    plsc.subcore_barrier()
    pltpu.sync_copy(acc.at[pl.ds(sid * RPT, RPT)],
                    out_hbm.at[cid, pl.ds(sid * RPT, RPT)])


def _elu(v):
    return jnp.where(v > 0.0, v, jnp.exp(jnp.minimum(v, 0.0)) - 1.0)


_MB = 2000
_GRID = N // _MB


def _tc_pre_body(degp_ref, x_ref, w_ref, dis_ref, g_ref):
    deg = degp_ref[0] + degp_ref[1] + 1.0
    dis = lax.rsqrt(deg)
    dis_ref[...] = dis
    h = jnp.dot(x_ref[...], w_ref[...], preferred_element_type=jnp.float32)
    g_ref[...] = h * dis


def _tc_pre(degp, x, w):
    return pl.pallas_call(
        _tc_pre_body,
        grid=(_GRID,),
        in_specs=[
            pl.BlockSpec((2, _MB, 1), lambda i: (0, i, 0)),
            pl.BlockSpec((_MB, D), lambda i: (i, 0)),
            pl.BlockSpec((D, D), lambda i: (0, 0)),
        ],
        out_specs=[
            pl.BlockSpec((_MB, 1), lambda i: (i, 0)),
            pl.BlockSpec((_MB, D), lambda i: (i, 0)),
        ],
        out_shape=[
            jax.ShapeDtypeStruct((N, 1), jnp.float32),
            jax.ShapeDtypeStruct((N, D), jnp.float32),
        ],
    )(degp, x, w)


def _tc_mid_body(p_ref, g_ref, dis_ref, b_ref, w_ref, out_ref):
    s = p_ref[0] + p_ref[1] + g_ref[...]
    h = _elu(dis_ref[...] * s + b_ref[...])
    out_ref[...] = jnp.dot(h, w_ref[...],
                           preferred_element_type=jnp.float32) * dis_ref[...]


def _tc_mid(p, g, dis, b, w):
    return pl.pallas_call(
        _tc_mid_body,
        grid=(_GRID,),
        in_specs=[
            pl.BlockSpec((2, _MB, D), lambda i: (0, i, 0)),
            pl.BlockSpec((_MB, D), lambda i: (i, 0)),
            pl.BlockSpec((_MB, 1), lambda i: (i, 0)),
            pl.BlockSpec((1, D), lambda i: (0, 0)),
            pl.BlockSpec((D, D), lambda i: (0, 0)),
        ],
        out_specs=pl.BlockSpec((_MB, D), lambda i: (i, 0)),
        out_shape=jax.ShapeDtypeStruct((N, D), jnp.float32),
    )(p, g, dis, b, w)


def _tc_post_body(p_ref, g_ref, dis_ref, b_ref, out_ref):
    s = p_ref[0] + p_ref[1] + g_ref[...]
    out_ref[...] = _elu(dis_ref[...] * s + b_ref[...])


def _tc_post(p, g, dis, b):
    return pl.pallas_call(
        _tc_post_body,
        grid=(_GRID,),
        in_specs=[
            pl.BlockSpec((2, _MB, D), lambda i: (0, i, 0)),
            pl.BlockSpec((_MB, D), lambda i: (i, 0)),
            pl.BlockSpec((_MB, 1), lambda i: (i, 0)),
            pl.BlockSpec((1, D), lambda i: (0, 0)),
        ],
        out_specs=pl.BlockSpec((_MB, D), lambda i: (i, 0)),
        out_shape=jax.ShapeDtypeStruct((N, D), jnp.float32),
    )(p, g, dis, b)


def kernel(x, edge_index, W1, b1, W2, b2):
    e = edge_index.astype(jnp.int32)
    src3 = e[0].reshape(NW, NCH, C)
    dst3 = e[1].reshape(NW, NCH, C)
    dstf = e[1].reshape(NW, EPT)

    degp = _sc_deg(dstf).reshape(2, NPAD, 1)
    dis, g1 = _tc_pre(degp, x, W1)
    p1 = _sc_scatter(g1, src3, dst3)
    g2 = _tc_mid(p1, g1, dis, b1.reshape(1, D), W2)
    p2 = _sc_scatter(g2, src3, dst3)
    return _tc_post(p2, g2, dis, b2.reshape(1, D))


# speedup vs baseline: 1.2877x; 1.0613x over previous
import functools

import jax
import jax.numpy as jnp
from jax import lax
from jax.experimental import pallas as pl
from jax.experimental.pallas import tpu as pltpu
from jax.experimental.pallas import tpu_sc as plsc

N = 10000
D = 128
NCH = 80
C = 125
EPT = NCH * C
NW = 32
NPAD = 10240
RPT = NPAD // 16

_mesh = plsc.VectorSubcoreMesh(core_axis_name="c", subcore_axis_name="s")


@functools.partial(
    pl.kernel,
    out_type=jax.ShapeDtypeStruct((2, NPAD), jnp.float32),
    mesh=_mesh,
    scratch_types=[
        pltpu.VMEM((EPT,), jnp.int32),
        pltpu.VMEM((NPAD,), jnp.float32),
        pltpu.VMEM((RPT,), jnp.float32),
        pltpu.VMEM((RPT,), jnp.float32),
        pltpu.VMEM_SHARED((16, NPAD), jnp.float32),
        pltpu.SemaphoreType.DMA,
    ],
    compiler_params=pltpu.CompilerParams(needs_layout_passes=False),
)
def _sc_deg(dst_hbm, out_hbm, dst_v, hist, red, tmp, shared, sem):
    cid = lax.axis_index("c")
    sid = lax.axis_index("s")
    wid = sid * 2 + cid
    pltpu.sync_copy(dst_hbm.at[wid], dst_v)
    zero = jnp.zeros((16,), jnp.float32)
    ones = jnp.ones((16,), jnp.float32)

    def z(i, carry):
        hist[pl.ds(i * 16, 16)] = zero
        return carry

    lax.fori_loop(0, NPAD // 16, z, None)

    def body(j, carry):
        idx = dst_v[pl.ds(j * 16, 16)]
        plsc.addupdate_scatter(hist, [idx], ones)
        return carry

    lax.fori_loop(0, EPT // 16, body, None)

    pltpu.sync_copy(hist, shared.at[sid])
    plsc.subcore_barrier()

    def zr(i, carry):
        red[pl.ds(i * 16, 16)] = zero
        return carry

    lax.fori_loop(0, RPT // 16, zr, None)

    def acc_t(t, carry):
        pltpu.sync_copy(shared.at[t, pl.ds(sid * RPT, RPT)], tmp)

        def addv(i, c2):
            s = pl.ds(i * 16, 16)
            red[s] = red[s] + tmp[s]
            return c2

        lax.fori_loop(0, RPT // 16, addv, None)
        return carry

    lax.fori_loop(0, 16, acc_t, None)
    pltpu.sync_copy(red, out_hbm.at[cid, pl.ds(sid * RPT, RPT)])


K = 2
UB = 20
NH = NCH // 2


@functools.partial(
    pl.kernel,
    out_type=jax.ShapeDtypeStruct((2, NPAD, D), jnp.float32),
    mesh=_mesh,
    scratch_types=[
        pltpu.VMEM((NH, C), jnp.int32),
        pltpu.VMEM((NH, C), jnp.int32),
        [pltpu.VMEM((C, D), jnp.float32) for _ in range(K)],
        pltpu.VMEM_SHARED((NPAD, D), jnp.float32),
        [pltpu.SemaphoreType.DMA for _ in range(K)],
    ],
)
def _sc_scatter(g_hbm, src_hbm, dst_hbm, out_hbm, src_v, dst_v, rows, acc,
                gsem):
    cid = lax.axis_index("c")
    sid = lax.axis_index("s")
    wid = sid * 2 + cid
    zero = jnp.zeros((16,), jnp.float32)

    def zrow(r, carry):
        for k in range(D // 16):
            rows[0][r, pl.ds(k * 16, 16)] = zero
        return carry

    lax.fori_loop(0, C, zrow, None)
    for k in range(RPT // C):
        pltpu.sync_copy(rows[0], acc.at[pl.ds(sid * RPT + k * C, C)])
    rem = RPT - (RPT // C) * C
    if rem:
        pltpu.sync_copy(rows[0].at[pl.ds(0, rem)],
                        acc.at[pl.ds(sid * RPT + (RPT // C) * C, rem)])
    plsc.subcore_barrier()

    for h in range(2):
        pltpu.sync_copy(src_hbm.at[wid, pl.ds(h * NH, NH)], src_v)
        pltpu.sync_copy(dst_hbm.at[wid, pl.ds(h * NH, NH)], dst_v)

        def body(i, carry):
            j = i * UB
            gd = {b: pltpu.async_copy(g_hbm.at[src_v.at[j + b]], rows[b],
                                      gsem[b]) for b in range(K)}
            for u in range(UB):
                b = u % K
                gd[u].wait()
                if u + K < UB:
                    gd[u + K] = pltpu.async_copy(
                        g_hbm.at[src_v.at[j + u + K]], rows[b], gsem[b])
                pltpu.sync_copy(rows[b], acc.at[dst_v.at[j + u]], add=True)
            return carry

        lax.fori_loop(0, NH // UB, body, None)
    plsc.subcore_barrier()
    pltpu.sync_copy(acc.at[pl.ds(sid * RPT, RPT)],
                    out_hbm.at[cid, pl.ds(sid * RPT, RPT)])


def _elu(v):
    return jnp.where(v > 0.0, v, jnp.exp(jnp.minimum(v, 0.0)) - 1.0)


_MB = 2000
_GRID = N // _MB


def _tc_pre_body(degp_ref, x_ref, w_ref, dis_ref, g_ref):
    deg = degp_ref[0] + degp_ref[1] + 1.0
    dis = lax.rsqrt(deg)
    dis_ref[...] = dis
    h = jnp.dot(x_ref[...], w_ref[...], preferred_element_type=jnp.float32)
    g_ref[...] = h * dis


def _tc_pre(degp, x, w):
    return pl.pallas_call(
        _tc_pre_body,
        grid=(_GRID,),
        in_specs=[
            pl.BlockSpec((2, _MB, 1), lambda i: (0, i, 0)),
            pl.BlockSpec((_MB, D), lambda i: (i, 0)),
            pl.BlockSpec((D, D), lambda i: (0, 0)),
        ],
        out_specs=[
            pl.BlockSpec((_MB, 1), lambda i: (i, 0)),
            pl.BlockSpec((_MB, D), lambda i: (i, 0)),
        ],
        out_shape=[
            jax.ShapeDtypeStruct((N, 1), jnp.float32),
            jax.ShapeDtypeStruct((N, D), jnp.float32),
        ],
    )(degp, x, w)


def _tc_mid_body(p_ref, g_ref, dis_ref, b_ref, w_ref, out_ref):
    s = p_ref[0] + p_ref[1] + g_ref[...]
    h = _elu(dis_ref[...] * s + b_ref[...])
    out_ref[...] = jnp.dot(h, w_ref[...],
                           preferred_element_type=jnp.float32) * dis_ref[...]


def _tc_mid(p, g, dis, b, w):
    return pl.pallas_call(
        _tc_mid_body,
        grid=(_GRID,),
        in_specs=[
            pl.BlockSpec((2, _MB, D), lambda i: (0, i, 0)),
            pl.BlockSpec((_MB, D), lambda i: (i, 0)),
            pl.BlockSpec((_MB, 1), lambda i: (i, 0)),
            pl.BlockSpec((1, D), lambda i: (0, 0)),
            pl.BlockSpec((D, D), lambda i: (0, 0)),
        ],
        out_specs=pl.BlockSpec((_MB, D), lambda i: (i, 0)),
        out_shape=jax.ShapeDtypeStruct((N, D), jnp.float32),
    )(p, g, dis, b, w)


def _tc_post_body(p_ref, g_ref, dis_ref, b_ref, out_ref):
    s = p_ref[0] + p_ref[1] + g_ref[...]
    out_ref[...] = _elu(dis_ref[...] * s + b_ref[...])


def _tc_post(p, g, dis, b):
    return pl.pallas_call(
        _tc_post_body,
        grid=(_GRID,),
        in_specs=[
            pl.BlockSpec((2, _MB, D), lambda i: (0, i, 0)),
            pl.BlockSpec((_MB, D), lambda i: (i, 0)),
            pl.BlockSpec((_MB, 1), lambda i: (i, 0)),
            pl.BlockSpec((1, D), lambda i: (0, 0)),
        ],
        out_specs=pl.BlockSpec((_MB, D), lambda i: (i, 0)),
        out_shape=jax.ShapeDtypeStruct((N, D), jnp.float32),
    )(p, g, dis, b)


def kernel(x, edge_index, W1, b1, W2, b2):
    e = edge_index.astype(jnp.int32)
    src3 = e[0].reshape(NW, NCH, C)
    dst3 = e[1].reshape(NW, NCH, C)
    dstf = e[1].reshape(NW, EPT)

    degp = _sc_deg(dstf).reshape(2, NPAD, 1)
    dis, g1 = _tc_pre(degp, x, W1)
    p1 = _sc_scatter(g1, src3, dst3)
    g2 = _tc_mid(p1, g1, dis, b1.reshape(1, D), W2)
    p2 = _sc_scatter(g2, src3, dst3)
    return _tc_post(p2, g2, dis, b2.reshape(1, D))


# speedup vs baseline: 1.3156x; 1.0217x over previous
import functools

import jax
import jax.numpy as jnp
from jax import lax
from jax.experimental import pallas as pl
from jax.experimental.pallas import tpu as pltpu
from jax.experimental.pallas import tpu_sc as plsc

N = 10000
D = 128
NCH = 80
C = 125
EPT = NCH * C
NW = 32
NPAD = 10240
RPT = NPAD // 16

_mesh = plsc.VectorSubcoreMesh(core_axis_name="c", subcore_axis_name="s")


@functools.partial(
    pl.kernel,
    out_type=jax.ShapeDtypeStruct((2, NPAD), jnp.float32),
    mesh=_mesh,
    scratch_types=[
        pltpu.VMEM((EPT,), jnp.int32),
        pltpu.VMEM((NPAD,), jnp.float32),
        pltpu.VMEM((RPT,), jnp.float32),
        pltpu.VMEM((RPT,), jnp.float32),
        pltpu.VMEM_SHARED((16, NPAD), jnp.float32),
        pltpu.SemaphoreType.DMA,
    ],
    compiler_params=pltpu.CompilerParams(needs_layout_passes=False),
)
def _sc_deg(dst_hbm, out_hbm, dst_v, hist, red, tmp, shared, sem):
    cid = lax.axis_index("c")
    sid = lax.axis_index("s")
    wid = sid * 2 + cid
    pltpu.sync_copy(dst_hbm.at[wid], dst_v)
    zero = jnp.zeros((16,), jnp.float32)
    ones = jnp.ones((16,), jnp.float32)

    def z(i, carry):
        hist[pl.ds(i * 16, 16)] = zero
        return carry

    lax.fori_loop(0, NPAD // 16, z, None)

    def body(j, carry):
        idx = dst_v[pl.ds(j * 16, 16)]
        plsc.addupdate_scatter(hist, [idx], ones)
        return carry

    lax.fori_loop(0, EPT // 16, body, None)

    pltpu.sync_copy(hist, shared.at[sid])
    plsc.subcore_barrier()

    def zr(i, carry):
        red[pl.ds(i * 16, 16)] = zero
        return carry

    lax.fori_loop(0, RPT // 16, zr, None)

    def acc_t(t, carry):
        pltpu.sync_copy(shared.at[t, pl.ds(sid * RPT, RPT)], tmp)

        def addv(i, c2):
            s = pl.ds(i * 16, 16)
            red[s] = red[s] + tmp[s]
            return c2

        lax.fori_loop(0, RPT // 16, addv, None)
        return carry

    lax.fori_loop(0, 16, acc_t, None)
    pltpu.sync_copy(red, out_hbm.at[cid, pl.ds(sid * RPT, RPT)])


K = 2
UB = 40
NH = NCH // 2


@functools.partial(
    pl.kernel,
    out_type=jax.ShapeDtypeStruct((2, NPAD, D), jnp.float32),
    mesh=_mesh,
    scratch_types=[
        pltpu.VMEM((NH, C), jnp.int32),
        pltpu.VMEM((NH, C), jnp.int32),
        [pltpu.VMEM((C, D), jnp.float32) for _ in range(K)],
        pltpu.VMEM_SHARED((NPAD, D), jnp.float32),
        [pltpu.SemaphoreType.DMA for _ in range(K)],
    ],
)
def _sc_scatter(g_hbm, src_hbm, dst_hbm, out_hbm, src_v, dst_v, rows, acc,
                gsem):
    cid = lax.axis_index("c")
    sid = lax.axis_index("s")
    wid = sid * 2 + cid
    zero = jnp.zeros((16,), jnp.float32)

    def zrow(r, carry):
        for k in range(D // 16):
            rows[0][r, pl.ds(k * 16, 16)] = zero
        return carry

    lax.fori_loop(0, C, zrow, None)
    for k in range(RPT // C):
        pltpu.sync_copy(rows[0], acc.at[pl.ds(sid * RPT + k * C, C)])
    rem = RPT - (RPT // C) * C
    if rem:
        pltpu.sync_copy(rows[0].at[pl.ds(0, rem)],
                        acc.at[pl.ds(sid * RPT + (RPT // C) * C, rem)])
    plsc.subcore_barrier()

    for h in range(2):
        pltpu.sync_copy(src_hbm.at[wid, pl.ds(h * NH, NH)], src_v)
        pltpu.sync_copy(dst_hbm.at[wid, pl.ds(h * NH, NH)], dst_v)

        def body(i, carry):
            j = i * UB
            gd = {b: pltpu.async_copy(g_hbm.at[src_v.at[j + b]], rows[b],
                                      gsem[b]) for b in range(K)}
            for u in range(UB):
                b = u % K
                gd[u].wait()
                if u + K < UB:
                    gd[u + K] = pltpu.async_copy(
                        g_hbm.at[src_v.at[j + u + K]], rows[b], gsem[b])
                pltpu.sync_copy(rows[b], acc.at[dst_v.at[j + u]], add=True)
            return carry

        lax.fori_loop(0, NH // UB, body, None)
    plsc.subcore_barrier()
    pltpu.sync_copy(acc.at[pl.ds(sid * RPT, RPT)],
                    out_hbm.at[cid, pl.ds(sid * RPT, RPT)])


def _elu(v):
    return jnp.where(v > 0.0, v, jnp.exp(jnp.minimum(v, 0.0)) - 1.0)


_MB = 2000
_GRID = N // _MB


def _tc_pre_body(degp_ref, x_ref, w_ref, dis_ref, g_ref):
    deg = degp_ref[0] + degp_ref[1] + 1.0
    dis = lax.rsqrt(deg)
    dis_ref[...] = dis
    h = jnp.dot(x_ref[...], w_ref[...], preferred_element_type=jnp.float32)
    g_ref[...] = h * dis


def _tc_pre(degp, x, w):
    return pl.pallas_call(
        _tc_pre_body,
        grid=(_GRID,),
        in_specs=[
            pl.BlockSpec((2, _MB, 1), lambda i: (0, i, 0)),
            pl.BlockSpec((_MB, D), lambda i: (i, 0)),
            pl.BlockSpec((D, D), lambda i: (0, 0)),
        ],
        out_specs=[
            pl.BlockSpec((_MB, 1), lambda i: (i, 0)),
            pl.BlockSpec((_MB, D), lambda i: (i, 0)),
        ],
        out_shape=[
            jax.ShapeDtypeStruct((N, 1), jnp.float32),
            jax.ShapeDtypeStruct((N, D), jnp.float32),
        ],
    )(degp, x, w)


def _tc_mid_body(p_ref, g_ref, dis_ref, b_ref, w_ref, out_ref):
    s = p_ref[0] + p_ref[1] + g_ref[...]
    h = _elu(dis_ref[...] * s + b_ref[...])
    out_ref[...] = jnp.dot(h, w_ref[...],
                           preferred_element_type=jnp.float32) * dis_ref[...]


def _tc_mid(p, g, dis, b, w):
    return pl.pallas_call(
        _tc_mid_body,
        grid=(_GRID,),
        in_specs=[
            pl.BlockSpec((2, _MB, D), lambda i: (0, i, 0)),
            pl.BlockSpec((_MB, D), lambda i: (i, 0)),
            pl.BlockSpec((_MB, 1), lambda i: (i, 0)),
            pl.BlockSpec((1, D), lambda i: (0, 0)),
            pl.BlockSpec((D, D), lambda i: (0, 0)),
        ],
        out_specs=pl.BlockSpec((_MB, D), lambda i: (i, 0)),
        out_shape=jax.ShapeDtypeStruct((N, D), jnp.float32),
    )(p, g, dis, b, w)


def _tc_post_body(p_ref, g_ref, dis_ref, b_ref, out_ref):
    s = p_ref[0] + p_ref[1] + g_ref[...]
    out_ref[...] = _elu(dis_ref[...] * s + b_ref[...])


def _tc_post(p, g, dis, b):
    return pl.pallas_call(
        _tc_post_body,
        grid=(_GRID,),
        in_specs=[
            pl.BlockSpec((2, _MB, D), lambda i: (0, i, 0)),
            pl.BlockSpec((_MB, D), lambda i: (i, 0)),
            pl.BlockSpec((_MB, 1), lambda i: (i, 0)),
            pl.BlockSpec((1, D), lambda i: (0, 0)),
        ],
        out_specs=pl.BlockSpec((_MB, D), lambda i: (i, 0)),
        out_shape=jax.ShapeDtypeStruct((N, D), jnp.float32),
    )(p, g, dis, b)


def kernel(x, edge_index, W1, b1, W2, b2):
    e = edge_index.astype(jnp.int32)
    src3 = e[0].reshape(NW, NCH, C)
    dst3 = e[1].reshape(NW, NCH, C)
    dstf = e[1].reshape(NW, EPT)

    degp = _sc_deg(dstf).reshape(2, NPAD, 1)
    dis, g1 = _tc_pre(degp, x, W1)
    p1 = _sc_scatter(g1, src3, dst3)
    g2 = _tc_mid(p1, g1, dis, b1.reshape(1, D), W2)
    p2 = _sc_scatter(g2, src3, dst3)
    return _tc_post(p2, g2, dis, b2.reshape(1, D))


# speedup vs baseline: 1.3219x; 1.0048x over previous
import functools

import jax
import jax.numpy as jnp
from jax import lax
from jax.experimental import pallas as pl
from jax.experimental.pallas import tpu as pltpu
from jax.experimental.pallas import tpu_sc as plsc

N = 10000
D = 128
NCH = 80
C = 125
EPT = NCH * C
NW = 32
NPAD = 10240
RPT = NPAD // 16

_mesh = plsc.VectorSubcoreMesh(core_axis_name="c", subcore_axis_name="s")


@functools.partial(
    pl.kernel,
    out_type=jax.ShapeDtypeStruct((2, NPAD), jnp.float32),
    mesh=_mesh,
    scratch_types=[
        pltpu.VMEM((EPT,), jnp.int32),
        pltpu.VMEM((NPAD,), jnp.float32),
        pltpu.VMEM((RPT,), jnp.float32),
        pltpu.VMEM((RPT,), jnp.float32),
        pltpu.VMEM_SHARED((16, NPAD), jnp.float32),
        pltpu.SemaphoreType.DMA,
    ],
    compiler_params=pltpu.CompilerParams(needs_layout_passes=False),
)
def _sc_deg(dst_hbm, out_hbm, dst_v, hist, red, tmp, shared, sem):
    cid = lax.axis_index("c")
    sid = lax.axis_index("s")
    wid = sid * 2 + cid
    pltpu.sync_copy(dst_hbm.at[wid], dst_v)
    zero = jnp.zeros((16,), jnp.float32)
    ones = jnp.ones((16,), jnp.float32)

    def z(i, carry):
        hist[pl.ds(i * 16, 16)] = zero
        return carry

    lax.fori_loop(0, NPAD // 16, z, None)

    def body(j, carry):
        idx = dst_v[pl.ds(j * 16, 16)]
        plsc.addupdate_scatter(hist, [idx], ones)
        return carry

    lax.fori_loop(0, EPT // 16, body, None)

    pltpu.sync_copy(hist, shared.at[sid])
    plsc.subcore_barrier()

    def zr(i, carry):
        red[pl.ds(i * 16, 16)] = zero
        return carry

    lax.fori_loop(0, RPT // 16, zr, None)

    def acc_t(t, carry):
        pltpu.sync_copy(shared.at[t, pl.ds(sid * RPT, RPT)], tmp)

        def addv(i, c2):
            s = pl.ds(i * 16, 16)
            red[s] = red[s] + tmp[s]
            return c2

        lax.fori_loop(0, RPT // 16, addv, None)
        return carry

    lax.fori_loop(0, 16, acc_t, None)
    pltpu.sync_copy(red, out_hbm.at[cid, pl.ds(sid * RPT, RPT)])


K = 2
UB = 40
NH = NCH // 2


@functools.partial(
    pl.kernel,
    out_type=jax.ShapeDtypeStruct((2, NPAD, D), jnp.float32),
    mesh=_mesh,
    scratch_types=[
        pltpu.VMEM((NH, C), jnp.int32),
        pltpu.VMEM((NH, C), jnp.int32),
        [pltpu.VMEM((C, D), jnp.float32) for _ in range(K)],
        pltpu.VMEM_SHARED((NPAD, D), jnp.float32),
        [pltpu.SemaphoreType.DMA for _ in range(K)],
    ],
)
def _sc_scatter(g_hbm, src_hbm, dst_hbm, out_hbm, src_v, dst_v, rows, acc,
                gsem):
    cid = lax.axis_index("c")
    sid = lax.axis_index("s")
    wid = sid * 2 + cid
    zero = jnp.zeros((16,), jnp.float32)

    def zrow(r, carry):
        for k in range(D // 16):
            rows[0][r, pl.ds(k * 16, 16)] = zero
        return carry

    lax.fori_loop(0, C, zrow, None)
    for k in range(RPT // C):
        pltpu.sync_copy(rows[0], acc.at[pl.ds(sid * RPT + k * C, C)])
    rem = RPT - (RPT // C) * C
    if rem:
        pltpu.sync_copy(rows[0].at[pl.ds(0, rem)],
                        acc.at[pl.ds(sid * RPT + (RPT // C) * C, rem)])
    plsc.subcore_barrier()

    for h in range(2):
        pltpu.sync_copy(src_hbm.at[wid, pl.ds(h * NH, NH)], src_v)
        pltpu.sync_copy(dst_hbm.at[wid, pl.ds(h * NH, NH)], dst_v)

        def body(i, carry):
            j = i * UB
            gd = {b: pltpu.async_copy(g_hbm.at[src_v.at[j + b]], rows[b],
                                      gsem[b]) for b in range(K)}
            for u in range(UB):
                b = u % K
                gd[u].wait()
                if u + K < UB:
                    gd[u + K] = pltpu.async_copy(
                        g_hbm.at[src_v.at[j + u + K]], rows[b], gsem[b])
                pltpu.sync_copy(rows[b], acc.at[dst_v.at[j + u]], add=True)
            return carry

        lax.fori_loop(0, NH // UB, body, None)
    plsc.subcore_barrier()
    pltpu.sync_copy(acc.at[pl.ds(sid * RPT, RPT)],
                    out_hbm.at[cid, pl.ds(sid * RPT, RPT)])


def _elu(v):
    return jnp.where(v > 0.0, v, jnp.exp(jnp.minimum(v, 0.0)) - 1.0)


_MB = 2000
_GRID = N // _MB


def _tc_mm_body(x_ref, w_ref, h_ref):
    h_ref[...] = jnp.dot(x_ref[...], w_ref[...],
                         preferred_element_type=jnp.float32)


def _tc_mm(x, w):
    return pl.pallas_call(
        _tc_mm_body,
        grid=(_GRID,),
        in_specs=[
            pl.BlockSpec((_MB, D), lambda i: (i, 0)),
            pl.BlockSpec((D, D), lambda i: (0, 0)),
        ],
        out_specs=pl.BlockSpec((_MB, D), lambda i: (i, 0)),
        out_shape=jax.ShapeDtypeStruct((N, D), jnp.float32),
    )(x, w)


def _tc_pre_body(degp_ref, h_ref, dis_ref, g_ref):
    deg = degp_ref[0] + degp_ref[1] + 1.0
    dis = lax.rsqrt(deg)
    dis_ref[...] = dis
    g_ref[...] = h_ref[...] * dis


def _tc_pre(degp, h):
    return pl.pallas_call(
        _tc_pre_body,
        grid=(_GRID,),
        in_specs=[
            pl.BlockSpec((2, _MB, 1), lambda i: (0, i, 0)),
            pl.BlockSpec((_MB, D), lambda i: (i, 0)),
        ],
        out_specs=[
            pl.BlockSpec((_MB, 1), lambda i: (i, 0)),
            pl.BlockSpec((_MB, D), lambda i: (i, 0)),
        ],
        out_shape=[
            jax.ShapeDtypeStruct((N, 1), jnp.float32),
            jax.ShapeDtypeStruct((N, D), jnp.float32),
        ],
    )(degp, h)


def _tc_mid_body(p_ref, g_ref, dis_ref, b_ref, w_ref, out_ref):
    s = p_ref[0] + p_ref[1] + g_ref[...]
    h = _elu(dis_ref[...] * s + b_ref[...])
    out_ref[...] = jnp.dot(h, w_ref[...],
                           preferred_element_type=jnp.float32) * dis_ref[...]


def _tc_mid(p, g, dis, b, w):
    return pl.pallas_call(
        _tc_mid_body,
        grid=(_GRID,),
        in_specs=[
            pl.BlockSpec((2, _MB, D), lambda i: (0, i, 0)),
            pl.BlockSpec((_MB, D), lambda i: (i, 0)),
            pl.BlockSpec((_MB, 1), lambda i: (i, 0)),
            pl.BlockSpec((1, D), lambda i: (0, 0)),
            pl.BlockSpec((D, D), lambda i: (0, 0)),
        ],
        out_specs=pl.BlockSpec((_MB, D), lambda i: (i, 0)),
        out_shape=jax.ShapeDtypeStruct((N, D), jnp.float32),
    )(p, g, dis, b, w)


def _tc_post_body(p_ref, g_ref, dis_ref, b_ref, out_ref):
    s = p_ref[0] + p_ref[1] + g_ref[...]
    out_ref[...] = _elu(dis_ref[...] * s + b_ref[...])


def _tc_post(p, g, dis, b):
    return pl.pallas_call(
        _tc_post_body,
        grid=(_GRID,),
        in_specs=[
            pl.BlockSpec((2, _MB, D), lambda i: (0, i, 0)),
            pl.BlockSpec((_MB, D), lambda i: (i, 0)),
            pl.BlockSpec((_MB, 1), lambda i: (i, 0)),
            pl.BlockSpec((1, D), lambda i: (0, 0)),
        ],
        out_specs=pl.BlockSpec((_MB, D), lambda i: (i, 0)),
        out_shape=jax.ShapeDtypeStruct((N, D), jnp.float32),
    )(p, g, dis, b)


def kernel(x, edge_index, W1, b1, W2, b2):
    e = edge_index.astype(jnp.int32)
    src3 = e[0].reshape(NW, NCH, C)
    dst3 = e[1].reshape(NW, NCH, C)
    dstf = e[1].reshape(NW, EPT)

    degp = _sc_deg(dstf).reshape(2, NPAD, 1)
    h1 = _tc_mm(x, W1)
    dis, g1 = _tc_pre(degp, h1)
    p1 = _sc_scatter(g1, src3, dst3)
    g2 = _tc_mid(p1, g1, dis, b1.reshape(1, D), W2)
    p2 = _sc_scatter(g2, src3, dst3)
    return _tc_post(p2, g2, dis, b2.reshape(1, D))
